# baseline (speedup 1.0000x reference)
import functools

import jax
import jax.numpy as jnp
from jax import lax
from jax.experimental import pallas as pl
from jax.experimental.pallas import tpu as pltpu
from jax.experimental.pallas import tpu_sc as plsc

EMBED = 64
SCALE = 8.0
NC, NS, L = 2, 16, 16
NW = NC * NS
CHUNK = 512


@functools.lru_cache(maxsize=None)
def _build(B: int):
    b_per_w = B // NW
    nchunks = b_per_w // CHUNK
    mesh = plsc.VectorSubcoreMesh(core_axis_name="c", subcore_axis_name="s")

    @functools.partial(
        pl.kernel,
        mesh=mesh,
        out_type=jax.ShapeDtypeStruct((B, EMBED), jnp.float32),
        scratch_types=[
            pltpu.VMEM((b_per_w,), jnp.int32),
            pltpu.VMEM((CHUNK, EMBED), jnp.float32),
            pltpu.SemaphoreType.DMA,
        ],
        compiler_params=pltpu.CompilerParams(use_tc_tiling_on_sc=False),
    )
    def emb(tok_hbm, table_hbm, out_hbm, idx_v, buf, sem):
        wid = lax.axis_index("s") * NC + lax.axis_index("c")
        base = wid * b_per_w
        pltpu.sync_copy(tok_hbm.at[pl.ds(base, b_per_w)], idx_v)

        def chunk_body(c, carry):
            off = c * CHUNK
            pltpu.async_copy(
                table_hbm.at[idx_v.at[pl.ds(off, CHUNK)]], buf, sem
            ).wait()

            def mul_body(i, carry2):
                for j in range(EMBED // L):
                    buf[i, pl.ds(j * L, L)] = buf[i, pl.ds(j * L, L)] * SCALE
                return carry2

            lax.fori_loop(0, CHUNK, mul_body, 0)
            pltpu.sync_copy(buf, out_hbm.at[pl.ds(base + off, CHUNK)])
            return carry

        lax.fori_loop(0, nchunks, chunk_body, 0)

    return emb


def kernel(tokens, embedding_weight):
    B = tokens.shape[0] * tokens.shape[1]
    flat = tokens.reshape(B).astype(jnp.int32)
    out = _build(B)(flat, embedding_weight)
    return out.reshape(tokens.shape + (EMBED,))


# speedup vs baseline: 1.1170x; 1.1170x over previous
import functools

import jax
import jax.numpy as jnp
from jax import lax
from jax.experimental import pallas as pl
from jax.experimental.pallas import tpu as pltpu
from jax.experimental.pallas import tpu_sc as plsc

EMBED = 64
SCALE = 8.0
NC, NS, L = 2, 16, 16
NW = NC * NS
CHUNK = 320
NBUF = 2
RU = 8


@functools.lru_cache(maxsize=None)
def _build(B: int):
    b_per_w = B // NW
    nchunks = b_per_w // CHUNK
    rounds = nchunks // NBUF
    mesh = plsc.VectorSubcoreMesh(core_axis_name="c", subcore_axis_name="s")

    @functools.partial(
        pl.kernel,
        mesh=mesh,
        out_type=jax.ShapeDtypeStruct((B, EMBED), jnp.float32),
        scratch_types=[
            pltpu.VMEM((b_per_w,), jnp.int32),
            [pltpu.VMEM((CHUNK, EMBED), jnp.float32) for _ in range(NBUF)],
            [pltpu.VMEM((CHUNK, EMBED), jnp.float32) for _ in range(NBUF)],
            [pltpu.SemaphoreType.DMA for _ in range(NBUF)],
            [pltpu.SemaphoreType.DMA for _ in range(NBUF)],
        ],
        compiler_params=pltpu.CompilerParams(use_tc_tiling_on_sc=False),
    )
    def emb(tok_hbm, table_hbm, out_hbm, idx_v, gbufs, sbufs, gsems, ssems):
        wid = lax.axis_index("s") * NC + lax.axis_index("c")
        base = wid * b_per_w
        pltpu.sync_copy(tok_hbm.at[pl.ds(base, b_per_w)], idx_v)

        for b in range(NBUF):
            pltpu.async_copy(
                table_hbm.at[idx_v.at[pl.ds(b * CHUNK, CHUNK)]],
                gbufs[b], gsems[b],
            )

        def round_body(g, carry):
            for b in range(NBUF):
                off = (g * NBUF + b) * CHUNK
                gbuf, sbuf = gbufs[b], sbufs[b]
                pltpu.make_async_copy(
                    table_hbm.at[idx_v.at[pl.ds(off, CHUNK)]], gbuf, gsems[b]
                ).wait()

                @pl.when(g > 0)
                def _():
                    pltpu.make_async_copy(
                        sbuf, out_hbm.at[pl.ds(base, CHUNK)], ssems[b]
                    ).wait()

                def mul_body(i, c2):
                    r0 = i * RU
                    for u in range(RU):
                        for j in range(EMBED // L):
                            sl = pl.ds(j * L, L)
                            sbuf[r0 + u, sl] = gbuf[r0 + u, sl] * SCALE
                    return c2

                lax.fori_loop(0, CHUNK // RU, mul_body, 0)

                @pl.when(g < rounds - 1)
                def _():
                    pltpu.async_copy(
                        table_hbm.at[
                            idx_v.at[pl.ds(off + NBUF * CHUNK, CHUNK)]
                        ],
                        gbuf, gsems[b],
                    )

                pltpu.async_copy(
                    sbuf, out_hbm.at[pl.ds(base + off, CHUNK)], ssems[b]
                )
            return carry

        lax.fori_loop(0, rounds, round_body, 0)

        for b in range(NBUF):
            pltpu.make_async_copy(
                sbufs[b], out_hbm.at[pl.ds(base, CHUNK)], ssems[b]
            ).wait()

    return emb


def kernel(tokens, embedding_weight):
    B = tokens.shape[0] * tokens.shape[1]
    flat = tokens.reshape(B).astype(jnp.int32)
    out = _build(B)(flat, embedding_weight)
    return out.reshape(tokens.shape + (EMBED,))
